# baseline (speedup 1.0000x reference)
import jax
import jax.numpy as jnp
from jax import lax
from jax.experimental import pallas as pl
from jax.experimental.pallas import tpu as pltpu
from jax.experimental.pallas import tpu_sc as plsc

_K = 1024
_D = 256
_BLK = 256
_N = 16384

_NC = 2
_NS = 16
_NW = _NC * _NS
_ROWS_PER_W = _N // _NW
_CHUNK = 256


def _vq_block(x_ref, wt_ref, w2_ref, idx_ref, enc_ref, loss_ref):
    i = pl.program_id(0)
    xb = x_ref[...]
    s = jnp.dot(xb, wt_ref[...], preferred_element_type=jnp.float32)
    x2 = jnp.sum(xb * xb, axis=1, keepdims=True)
    dist = (x2 + w2_ref[...]) - 2.0 * s
    m = jnp.min(dist, axis=1, keepdims=True)
    iota = jax.lax.broadcasted_iota(jnp.int32, (_BLK, _K), 1)
    idx = jnp.min(jnp.where(dist == m, iota, _K), axis=1)
    enc_ref[...] = (iota == idx[:, None]).astype(jnp.float32)
    idx_ref[...] = idx[:, None]

    part = jnp.sum(m).reshape(1, 1)

    @pl.when(i == 0)
    def _init():
        loss_ref[...] = jnp.zeros((1, 1), jnp.float32)

    loss_ref[...] += part


def _gather_body(w_hbm, idx_hbm, out_hbm, idx_v, rows_v, sem):
    wid = lax.axis_index("s") * _NC + lax.axis_index("c")
    base = wid * _ROWS_PER_W
    for c in range(_ROWS_PER_W // _CHUNK):
        off = base + c * _CHUNK
        pltpu.sync_copy(idx_hbm.at[pl.ds(off, _CHUNK)], idx_v)
        pltpu.async_copy(w_hbm.at[idx_v], rows_v, sem).wait()
        pltpu.sync_copy(rows_v, out_hbm.at[pl.ds(off, _CHUNK)])


_gather_rows = pl.kernel(
    _gather_body,
    out_type=jax.ShapeDtypeStruct((_N, _D), jnp.float32),
    mesh=plsc.VectorSubcoreMesh(core_axis_name="c", subcore_axis_name="s"),
    scratch_types=[
        pltpu.VMEM((_CHUNK,), jnp.int32),
        pltpu.VMEM((_CHUNK, _D), jnp.float32),
        pltpu.SemaphoreType.DMA,
    ],
)


def kernel(x, W):
    B, C, H, Wd = x.shape
    flat_x = jnp.transpose(x, (0, 2, 3, 1)).reshape(-1, _D)
    n = flat_x.shape[0]
    wt = W.T
    w2 = jnp.sum(W ** 2, axis=1)[None, :]

    idx2, enc, loss_sum = pl.pallas_call(
        _vq_block,
        grid=(n // _BLK,),
        in_specs=[
            pl.BlockSpec((_BLK, _D), lambda i: (i, 0)),
            pl.BlockSpec((_D, _K), lambda i: (0, 0)),
            pl.BlockSpec((1, _K), lambda i: (0, 0)),
        ],
        out_specs=[
            pl.BlockSpec((_BLK, 1), lambda i: (i, 0)),
            pl.BlockSpec((_BLK, _K), lambda i: (i, 0)),
            pl.BlockSpec((1, 1), lambda i: (0, 0)),
        ],
        out_shape=[
            jax.ShapeDtypeStruct((n, 1), jnp.int32),
            jax.ShapeDtypeStruct((n, _K), jnp.float32),
            jax.ShapeDtypeStruct((1, 1), jnp.float32),
        ],
    )(flat_x, wt, w2)

    idx_flat = idx2.reshape(-1)
    q = _gather_rows(W, idx_flat)

    loss = 2.0 * loss_sum[0, 0] / (n * _D)
    out = jnp.transpose(q.reshape(B, H, Wd, C), (0, 3, 1, 2))
    return (loss, out, enc, idx_flat)


# speedup vs baseline: 1.1167x; 1.1167x over previous
import jax
import jax.numpy as jnp
from jax.experimental import pallas as pl

_K = 1024
_D = 256
_RT = 256
_N = 16384


def _vq_block(x3_ref, w_ref, w2c_ref, idx_ref, enc_ref, q_ref, loss_ref):
    i = pl.program_id(0)
    xb = x3_ref[0]
    s = jax.lax.dot_general(w_ref[...], xb, (((1,), (0,)), ((), ())),
                            preferred_element_type=jnp.float32)
    x2 = jnp.sum(xb * xb, axis=0, keepdims=True)
    dist = (x2 + w2c_ref[...]) - 2.0 * s
    m = jnp.min(dist, axis=0, keepdims=True)
    iota_k = jax.lax.broadcasted_iota(jnp.int32, (_K, _RT), 0)
    idxv = jnp.min(jnp.where(dist == m, iota_k, _K), axis=0)
    idx_ref[...] = idxv[None, None, :]

    idxc = idxv[:, None]
    iota_r = jax.lax.broadcasted_iota(jnp.int32, (_RT, _K), 1)
    enc = (iota_r == idxc).astype(jnp.float32)
    enc_ref[...] = enc
    q_ref[...] = jnp.dot(enc, w_ref[...],
                         preferred_element_type=jnp.float32)

    part = jnp.sum(m).reshape(1, 1)

    @pl.when(i == 0)
    def _init():
        loss_ref[...] = jnp.zeros((1, 1), jnp.float32)

    loss_ref[...] += part


def kernel(x, W):
    B, C, H, Wd = x.shape
    x3 = x.reshape(B, C, H * Wd)
    n = B * H * Wd
    w2c = jnp.sum(W ** 2, axis=1)[:, None]
    nt = n // _RT
    rpb = H * Wd // _RT

    idx3, enc, q, loss_sum = pl.pallas_call(
        _vq_block,
        grid=(nt,),
        in_specs=[
            pl.BlockSpec((1, _D, _RT), lambda i: (i // rpb, 0, i % rpb)),
            pl.BlockSpec((_K, _D), lambda i: (0, 0)),
            pl.BlockSpec((_K, 1), lambda i: (0, 0)),
        ],
        out_specs=[
            pl.BlockSpec((1, 1, _RT), lambda i: (i, 0, 0)),
            pl.BlockSpec((_RT, _K), lambda i: (i, 0)),
            pl.BlockSpec((_RT, _D), lambda i: (i, 0)),
            pl.BlockSpec((1, 1), lambda i: (0, 0)),
        ],
        out_shape=[
            jax.ShapeDtypeStruct((nt, 1, _RT), jnp.int32),
            jax.ShapeDtypeStruct((n, _K), jnp.float32),
            jax.ShapeDtypeStruct((n, _D), jnp.float32),
            jax.ShapeDtypeStruct((1, 1), jnp.float32),
        ],
    )(x3, W, w2c)

    loss = 2.0 * loss_sum[0, 0] / (n * _D)
    out = jnp.transpose(q.reshape(B, H, Wd, C), (0, 3, 1, 2))
    return (loss, out, enc, idx3.reshape(-1))
